# Initial kernel scaffold; baseline (speedup 1.0000x reference)
import jax
import jax.numpy as jnp
from jax.experimental import pallas as pl


def kernel(x, embeddings, W, b):
    raise NotImplementedError("write your pallas kernel here")


# speedup vs baseline: 2.1150x; 2.1150x over previous
import functools

import jax
import jax.numpy as jnp
from jax import lax
from jax.experimental import pallas as pl
from jax.experimental.pallas import tpu as pltpu
from jax.experimental.pallas import tpu_sc as plsc

BATCH = 16384
HIST = 50
EMBED_DIM = 64
LANES = 16
NUM_CORES = 2
NUM_SUBCORES = 16
NUM_WORKERS = NUM_CORES * NUM_SUBCORES
SAMPLES_PER_WORKER = BATCH // NUM_WORKERS
CHUNK_SAMPLES = 2
CHUNK_ROWS = CHUNK_SAMPLES * HIST
NUM_CHUNKS = SAMPLES_PER_WORKER // CHUNK_SAMPLES
VREGS_PER_ROW = EMBED_DIM // LANES


def _sc_gather_sum(x_hbm, tab_hbm, out_hbm, idx_v, rows_v, acc_v, sem):
    wid = lax.axis_index("s") * NUM_CORES + lax.axis_index("c")

    pltpu.sync_copy(x_hbm.at[wid], idx_v)

    def chunk_body(c, _):
        pltpu.async_copy(tab_hbm.at[idx_v.at[c]], rows_v, sem).wait()
        for s in range(CHUNK_SAMPLES):
            base = s * HIST

            def red(l, acc):
                return tuple(
                    acc[j] + rows_v[base + l, pl.ds(j * LANES, LANES)]
                    for j in range(VREGS_PER_ROW)
                )

            acc = tuple(
                rows_v[base, pl.ds(j * LANES, LANES)]
                for j in range(VREGS_PER_ROW)
            )
            acc = lax.fori_loop(1, HIST, red, acc)
            for j in range(VREGS_PER_ROW):
                acc_v[c * CHUNK_SAMPLES + s, pl.ds(j * LANES, LANES)] = acc[j]
        return 0

    lax.fori_loop(0, NUM_CHUNKS, chunk_body, 0)

    pltpu.sync_copy(acc_v, out_hbm.at[wid])


@functools.partial(
    pl.kernel,
    out_type=jax.ShapeDtypeStruct(
        (NUM_WORKERS, SAMPLES_PER_WORKER, EMBED_DIM), jnp.float32
    ),
    mesh=plsc.VectorSubcoreMesh(core_axis_name="c", subcore_axis_name="s"),
    compiler_params=pltpu.CompilerParams(use_tc_tiling_on_sc=False),
    scratch_types=[
        pltpu.VMEM((NUM_CHUNKS, CHUNK_ROWS), jnp.int32),
        pltpu.VMEM((CHUNK_ROWS, EMBED_DIM), jnp.float32),
        pltpu.VMEM((SAMPLES_PER_WORKER, EMBED_DIM), jnp.float32),
        pltpu.SemaphoreType.DMA,
    ],
)
def _gather_sum_call(x_hbm, tab_hbm, out_hbm, idx_v, rows_v, acc_v, sem):
    _sc_gather_sum(x_hbm, tab_hbm, out_hbm, idx_v, rows_v, acc_v, sem)


def _linear_body(x_ref, w_ref, b_ref, o_ref):
    o_ref[...] = (
        jnp.dot(x_ref[...], w_ref[...], preferred_element_type=jnp.float32)
        * (1.0 / HIST)
        + b_ref[...]
    )


def _linear(t_sum, W, b):
    blk = 2048
    return pl.pallas_call(
        _linear_body,
        grid=(BATCH // blk,),
        in_specs=[
            pl.BlockSpec((blk, EMBED_DIM), lambda i: (i, 0)),
            pl.BlockSpec((EMBED_DIM, 2), lambda i: (0, 0)),
            pl.BlockSpec((1, 2), lambda i: (0, 0)),
        ],
        out_specs=pl.BlockSpec((blk, 2), lambda i: (i, 0)),
        out_shape=jax.ShapeDtypeStruct((BATCH, 2), jnp.float32),
    )(t_sum, W, b.reshape(1, 2))


@jax.jit
def kernel(x, embeddings, W, b):
    idx = x.astype(jnp.int32).reshape(NUM_WORKERS, NUM_CHUNKS, CHUNK_ROWS)
    t_sum = _gather_sum_call(idx, embeddings)
    t_sum = t_sum.reshape(BATCH, EMBED_DIM)
    return _linear(t_sum, W, b)


# speedup vs baseline: 8.8185x; 4.1695x over previous
import functools

import jax
import jax.numpy as jnp
from jax import lax
from jax.experimental import pallas as pl
from jax.experimental.pallas import tpu as pltpu
from jax.experimental.pallas import tpu_sc as plsc

BATCH = 16384
HIST = 50
EMBED_DIM = 64
VOCAB = 1000000
LANES = 16
NUM_CORES = 2
NUM_SUBCORES = 16
NUM_WORKERS = NUM_CORES * NUM_SUBCORES
SPT = BATCH // NUM_WORKERS
KSUB = SPT // 128
PROJ_CHUNK = 32768


def _proj_body(emb_t_ref, wt_ref, bp_ref, o0_ref, o1_ref):
    x = emb_t_ref[...]
    w = wt_ref[...]
    bp = bp_ref[...]
    o0_ref[...] = jnp.sum(x * w[0][:, None], axis=0) + bp[0, 0]
    o1_ref[...] = jnp.sum(x * w[1][:, None], axis=0) + bp[1, 0]


def _project(emb_t, wt, bp):
    return pl.pallas_call(
        _proj_body,
        grid=(pl.cdiv(VOCAB, PROJ_CHUNK),),
        in_specs=[
            pl.BlockSpec((EMBED_DIM, PROJ_CHUNK), lambda i: (0, i)),
            pl.BlockSpec((2, EMBED_DIM), lambda i: (0, 0)),
            pl.BlockSpec((2, 1), lambda i: (0, 0)),
        ],
        out_specs=[
            pl.BlockSpec((PROJ_CHUNK,), lambda i: (i,)),
            pl.BlockSpec((PROJ_CHUNK,), lambda i: (i,)),
        ],
        out_shape=[
            jax.ShapeDtypeStruct((VOCAB,), jnp.float32),
            jax.ShapeDtypeStruct((VOCAB,), jnp.float32),
        ],
    )(emb_t, wt, bp)


def _sc_gather_sum(
    xt_hbm, p0_hbm, p1_hbm, o0_hbm, o1_hbm, idx_v, val_v, acc0_v, acc1_v,
    sem_a, sem_b,
):
    wid = lax.axis_index("s") * NUM_CORES + lax.axis_index("c")
    base = wid * SPT

    pltpu.sync_copy(xt_hbm.at[wid], idx_v)

    zv = jnp.zeros((LANES,), jnp.float32)

    def zbody(i, _):
        acc0_v[pl.ds(i * LANES, LANES)] = zv
        acc1_v[pl.ds(i * LANES, LANES)] = zv
        return 0

    lax.fori_loop(0, SPT // LANES, zbody, 0)

    def lbody(l, _):
        copies = []
        for k in range(KSUB):
            idx_ref = idx_v.at[l, k]
            copies.append(
                pltpu.async_copy(p0_hbm.at[idx_ref], val_v.at[0, 0, k], sem_a)
            )
            copies.append(
                pltpu.async_copy(p1_hbm.at[idx_ref], val_v.at[0, 1, k], sem_b)
            )
        for c in copies:
            c.wait()
        for k in range(KSUB):
            for r in range(8):
                sl = pl.ds(k * 128 + r * LANES, LANES)
                vs = pl.ds(r * LANES, LANES)
                acc0_v[sl] = acc0_v[sl] + val_v[0, 0, k, vs]
                acc1_v[sl] = acc1_v[sl] + val_v[0, 1, k, vs]
        return 0

    lax.fori_loop(0, HIST, lbody, 0)

    pltpu.sync_copy(acc0_v, o0_hbm.at[pl.ds(base, SPT)])
    pltpu.sync_copy(acc1_v, o1_hbm.at[pl.ds(base, SPT)])


@functools.partial(
    pl.kernel,
    out_type=[
        jax.ShapeDtypeStruct((BATCH,), jnp.float32),
        jax.ShapeDtypeStruct((BATCH,), jnp.float32),
    ],
    mesh=plsc.VectorSubcoreMesh(core_axis_name="c", subcore_axis_name="s"),
    compiler_params=pltpu.CompilerParams(use_tc_tiling_on_sc=False),
    scratch_types=[
        pltpu.VMEM((HIST, KSUB, 128), jnp.int32),
        pltpu.VMEM((2, 2, KSUB, 128), jnp.float32),
        pltpu.VMEM((SPT,), jnp.float32),
        pltpu.VMEM((SPT,), jnp.float32),
        pltpu.SemaphoreType.DMA,
        pltpu.SemaphoreType.DMA,
    ],
)
def _gather_sum_call(
    xt_hbm, p0_hbm, p1_hbm, o0_hbm, o1_hbm, idx_v, val_v, acc0_v, acc1_v,
    sem_a, sem_b,
):
    _sc_gather_sum(
        xt_hbm, p0_hbm, p1_hbm, o0_hbm, o1_hbm, idx_v, val_v, acc0_v, acc1_v,
        sem_a, sem_b,
    )


@jax.jit
def kernel(x, embeddings, W, b):
    scale = 1.0 / HIST
    wt = jnp.transpose(W) * scale
    bp = (b * scale).reshape(2, 1)
    emb_t = jnp.transpose(embeddings)
    p0, p1 = _project(emb_t, wt, bp)
    xt = jnp.transpose(x.astype(jnp.int32))
    x_idx = xt.reshape(HIST, NUM_WORKERS, KSUB, 128).transpose(1, 0, 2, 3)
    o0, o1 = _gather_sum_call(x_idx, p0, p1)
    return jnp.stack([o0, o1], axis=1)


# speedup vs baseline: 9.5685x; 1.0851x over previous
import functools

import jax
import jax.numpy as jnp
from jax import lax
from jax.experimental import pallas as pl
from jax.experimental.pallas import tpu as pltpu
from jax.experimental.pallas import tpu_sc as plsc

BATCH = 16384
HIST = 50
EMBED_DIM = 64
VOCAB = 1000000
LANES = 16
NUM_CORES = 2
NUM_SUBCORES = 16
NUM_WORKERS = NUM_CORES * NUM_SUBCORES
SPT = BATCH // NUM_WORKERS
KSUB = SPT // 128
PROJ_CHUNK = 32768


def _proj_body(emb_t_ref, wt_ref, bp_ref, o0_ref, o1_ref):
    x = emb_t_ref[...]
    w = wt_ref[...]
    bp = bp_ref[...]
    o0_ref[...] = jnp.sum(x * w[0][:, None], axis=0) + bp[0, 0]
    o1_ref[...] = jnp.sum(x * w[1][:, None], axis=0) + bp[1, 0]


def _project(emb_t, wt, bp):
    return pl.pallas_call(
        _proj_body,
        grid=(pl.cdiv(VOCAB, PROJ_CHUNK),),
        in_specs=[
            pl.BlockSpec((EMBED_DIM, PROJ_CHUNK), lambda i: (0, i)),
            pl.BlockSpec((2, EMBED_DIM), lambda i: (0, 0)),
            pl.BlockSpec((2, 1), lambda i: (0, 0)),
        ],
        out_specs=[
            pl.BlockSpec((PROJ_CHUNK,), lambda i: (i,)),
            pl.BlockSpec((PROJ_CHUNK,), lambda i: (i,)),
        ],
        out_shape=[
            jax.ShapeDtypeStruct((VOCAB,), jnp.float32),
            jax.ShapeDtypeStruct((VOCAB,), jnp.float32),
        ],
    )(emb_t, wt, bp)


def _sc_gather_sum(
    xt_hbm, p0_hbm, p1_hbm, o0_hbm, o1_hbm, idx_v, val_v, acc0_v, acc1_v,
    sem_a, sem_b,
):
    wid = lax.axis_index("s") * NUM_CORES + lax.axis_index("c")
    base = wid * SPT
    sems = (sem_a, sem_b)

    pltpu.sync_copy(xt_hbm.at[wid], idx_v)

    def issue(l, buf):
        for k in range(KSUB):
            idx_ref = idx_v.at[l, k]
            pltpu.async_copy(p0_hbm.at[idx_ref], val_v.at[buf, 0, k], sems[buf])
            pltpu.async_copy(p1_hbm.at[idx_ref], val_v.at[buf, 1, k], sems[buf])

    def drain(l, buf):
        for k in range(KSUB):
            idx_ref = idx_v.at[l, k]
            pltpu.make_async_copy(
                p0_hbm.at[idx_ref], val_v.at[buf, 0, k], sems[buf]
            ).wait()
            pltpu.make_async_copy(
                p1_hbm.at[idx_ref], val_v.at[buf, 1, k], sems[buf]
            ).wait()

    issue(0, 0)

    zv = jnp.zeros((LANES,), jnp.float32)

    def zbody(i, _):
        acc0_v[pl.ds(i * LANES, LANES)] = zv
        acc1_v[pl.ds(i * LANES, LANES)] = zv
        return 0

    lax.fori_loop(0, SPT // LANES, zbody, 0)

    def accumulate(buf):
        for k in range(KSUB):
            for r in range(8):
                sl = pl.ds(k * 128 + r * LANES, LANES)
                vs = pl.ds(r * LANES, LANES)
                acc0_v[sl] = acc0_v[sl] + val_v[buf, 0, k, vs]
                acc1_v[sl] = acc1_v[sl] + val_v[buf, 1, k, vs]

    def lbody(j, _):
        for t in range(2):
            l = 2 * j + t

            @pl.when(l + 1 < HIST)
            def _():
                issue(l + 1, 1 - t)

            drain(l, t)
            accumulate(t)
        return 0

    lax.fori_loop(0, HIST // 2, lbody, 0)

    pltpu.sync_copy(acc0_v, o0_hbm.at[pl.ds(base, SPT)])
    pltpu.sync_copy(acc1_v, o1_hbm.at[pl.ds(base, SPT)])


@functools.partial(
    pl.kernel,
    out_type=[
        jax.ShapeDtypeStruct((BATCH,), jnp.float32),
        jax.ShapeDtypeStruct((BATCH,), jnp.float32),
    ],
    mesh=plsc.VectorSubcoreMesh(core_axis_name="c", subcore_axis_name="s"),
    compiler_params=pltpu.CompilerParams(use_tc_tiling_on_sc=False),
    scratch_types=[
        pltpu.VMEM((HIST, KSUB, 128), jnp.int32),
        pltpu.VMEM((2, 2, KSUB, 128), jnp.float32),
        pltpu.VMEM((SPT,), jnp.float32),
        pltpu.VMEM((SPT,), jnp.float32),
        pltpu.SemaphoreType.DMA,
        pltpu.SemaphoreType.DMA,
    ],
)
def _gather_sum_call(
    xt_hbm, p0_hbm, p1_hbm, o0_hbm, o1_hbm, idx_v, val_v, acc0_v, acc1_v,
    sem_a, sem_b,
):
    _sc_gather_sum(
        xt_hbm, p0_hbm, p1_hbm, o0_hbm, o1_hbm, idx_v, val_v, acc0_v, acc1_v,
        sem_a, sem_b,
    )


@jax.jit
def kernel(x, embeddings, W, b):
    scale = 1.0 / HIST
    wt = jnp.transpose(W) * scale
    bp = (b * scale).reshape(2, 1)
    emb_t = jnp.transpose(embeddings)
    p0, p1 = _project(emb_t, wt, bp)
    xt = jnp.transpose(x.astype(jnp.int32))
    x_idx = xt.reshape(HIST, NUM_WORKERS, KSUB, 128).transpose(1, 0, 2, 3)
    o0, o1 = _gather_sum_call(x_idx, p0, p1)
    return jnp.stack([o0, o1], axis=1)
